# baseline (speedup 1.0000x reference)
import functools

import jax
import jax.numpy as jnp
from jax import lax
from jax.experimental import pallas as pl
from jax.experimental.pallas import tpu as pltpu
from jax.experimental.pallas import tpu_sc as plsc

VOCAB = 100000
D_MODEL = 4096
BATCH = 4
SEQ = 8192

_B = BATCH * SEQ

_info = plsc.get_sparse_core_info()
_NC, _NS = _info.num_cores, _info.num_subcores
_NW = _NC * _NS
_B_PER_W = _B // _NW
_CHUNK = 4
_NITER = _B_PER_W // _CHUNK
_NBUF = 6


def _sc_embed(ids_flat, table):
    mesh = plsc.VectorSubcoreMesh(core_axis_name="c", subcore_axis_name="s")

    scratch = (
        [pltpu.VMEM((_NITER, _CHUNK), jnp.int32)]
        + [pltpu.VMEM((_CHUNK, D_MODEL), jnp.float32) for _ in range(_NBUF)]
        + [pltpu.SemaphoreType.DMA for _ in range(2 * _NBUF)]
    )

    @functools.partial(
        pl.kernel,
        mesh=mesh,
        out_type=jax.ShapeDtypeStruct((_B, D_MODEL), jnp.float32),
        scratch_types=scratch,
    )
    def k(ids_hbm, table_hbm, out_hbm, idx_v, *rest):
        bufs = rest[:_NBUF]
        gsem = rest[_NBUF:2 * _NBUF]
        ssem = rest[2 * _NBUF:]

        wid = lax.axis_index("s") * _NC + lax.axis_index("c")
        base = wid * _B_PER_W
        pltpu.sync_copy(ids_hbm.at[wid], idx_v)

        def g_start(c, b):
            pltpu.async_copy(table_hbm.at[idx_v.at[c]], bufs[b], gsem[b])

        def g_wait(b):
            pltpu.make_async_copy(
                table_hbm.at[idx_v.at[0]], bufs[b], gsem[b]
            ).wait()

        def s_start(c, b):
            pltpu.async_copy(
                bufs[b], out_hbm.at[pl.ds(base + c * _CHUNK, _CHUNK)], ssem[b]
            )

        def s_wait(b):
            pltpu.make_async_copy(
                bufs[b], out_hbm.at[pl.ds(base, _CHUNK)], ssem[b]
            ).wait()

        for b in range(_NBUF - 1):
            g_start(b, b)

        def step(c, b):
            g_wait(b)
            s_start(c, b)
            fg = c + (_NBUF - 1)
            fb = (b + _NBUF - 1) % _NBUF

            @pl.when(fg < _NITER)
            def _():
                @pl.when(fg >= _NBUF)
                def _():
                    s_wait(fb)

                g_start(fg, fb)

        def body(j, carry):
            for b in range(_NBUF):
                c = j * _NBUF + b

                @pl.when(c < _NITER)
                def _():
                    step(c, b)

            return carry

        lax.fori_loop(0, (_NITER + _NBUF - 1) // _NBUF, body, 0)

        for b in range(_NBUF):
            s_wait(b)

    return k(ids_flat, table)


def kernel(input_ids, table):
    ids = input_ids.reshape(_NW, _NITER, _CHUNK).astype(jnp.int32)
    out = _sc_embed(ids, table)
    return out.reshape(BATCH, SEQ, D_MODEL)


# speedup vs baseline: 1.0134x; 1.0134x over previous
import functools

import jax
import jax.numpy as jnp
from jax import lax
from jax.experimental import pallas as pl
from jax.experimental.pallas import tpu as pltpu
from jax.experimental.pallas import tpu_sc as plsc

VOCAB = 100000
D_MODEL = 4096
BATCH = 4
SEQ = 8192

_B = BATCH * SEQ

_info = plsc.get_sparse_core_info()
_NC, _NS = _info.num_cores, _info.num_subcores
_NW = _NC * _NS
_B_PER_W = _B // _NW
_CHUNK = 8
_NITER = _B_PER_W // _CHUNK
_NBUF = 3


def _sc_embed(ids_flat, table):
    mesh = plsc.VectorSubcoreMesh(core_axis_name="c", subcore_axis_name="s")

    scratch = (
        [pltpu.VMEM((_NITER, _CHUNK), jnp.int32)]
        + [pltpu.VMEM((_CHUNK, D_MODEL), jnp.float32) for _ in range(_NBUF)]
        + [pltpu.SemaphoreType.DMA for _ in range(2 * _NBUF)]
    )

    @functools.partial(
        pl.kernel,
        mesh=mesh,
        out_type=jax.ShapeDtypeStruct((_B, D_MODEL), jnp.float32),
        scratch_types=scratch,
    )
    def k(ids_hbm, table_hbm, out_hbm, idx_v, *rest):
        bufs = rest[:_NBUF]
        gsem = rest[_NBUF:2 * _NBUF]
        ssem = rest[2 * _NBUF:]

        wid = lax.axis_index("s") * _NC + lax.axis_index("c")
        base = wid * _B_PER_W
        pltpu.sync_copy(ids_hbm.at[wid], idx_v)

        def g_start(c, b):
            pltpu.async_copy(table_hbm.at[idx_v.at[c]], bufs[b], gsem[b])

        def g_wait(b):
            pltpu.make_async_copy(
                table_hbm.at[idx_v.at[0]], bufs[b], gsem[b]
            ).wait()

        def s_start(c, b):
            pltpu.async_copy(
                bufs[b], out_hbm.at[pl.ds(base + c * _CHUNK, _CHUNK)], ssem[b]
            )

        def s_wait(b):
            pltpu.make_async_copy(
                bufs[b], out_hbm.at[pl.ds(base, _CHUNK)], ssem[b]
            ).wait()

        for b in range(_NBUF - 1):
            g_start(b, b)

        def step(c, b):
            g_wait(b)
            fg = c + (_NBUF - 1)
            fb = (b + _NBUF - 1) % _NBUF

            @pl.when(fg < _NITER)
            def _():
                @pl.when(fg >= _NBUF)
                def _():
                    s_wait(fb)

                g_start(fg, fb)

            s_start(c, b)

        def body(j, carry):
            for b in range(_NBUF):
                c = j * _NBUF + b

                @pl.when(c < _NITER)
                def _():
                    step(c, b)

            return carry

        lax.fori_loop(0, (_NITER + _NBUF - 1) // _NBUF, body, 0)

        for b in range(_NBUF):
            s_wait(b)

    return k(ids_flat, table)


def kernel(input_ids, table):
    ids = input_ids.reshape(_NW, _NITER, _CHUNK).astype(jnp.int32)
    out = _sc_embed(ids, table)
    return out.reshape(BATCH, SEQ, D_MODEL)
